# baseline (speedup 1.0000x reference)
import functools

import jax
import jax.numpy as jnp
from jax import lax
from jax.experimental import pallas as pl
from jax.experimental.pallas import tpu as pltpu
from jax.experimental.pallas import tpu_sc as plsc

VOCAB = 100000
EMBED_DIM = 64
BATCH = 4096
SEQ = 50
N = BATCH * SEQ
HALF = N // 2

_info = plsc.get_sparse_core_info()
NC = _info.num_cores
NS = _info.num_subcores
NW = NC * NS
B_PER_W = HALF // NW
CHUNK = 128
NCHUNKS = B_PER_W // CHUNK
NBUF = 5


def _gather_body(x_hbm, table_hbm, out_hbm, idx_v, rows_v, gsems, wsems):
    wid = lax.axis_index("s") * NC + lax.axis_index("c")
    base = wid * B_PER_W
    pltpu.sync_copy(x_hbm.at[wid], idx_v)

    def start_gather(g, j):
        pltpu.make_async_copy(
            table_hbm.at[idx_v.at[g]], rows_v.at[j], gsems.at[j]
        ).start()

    def wait_gather(j):
        pltpu.make_async_copy(
            table_hbm.at[idx_v.at[0]], rows_v.at[j], gsems.at[j]
        ).wait()

    def start_write(g, j):
        off = pl.multiple_of(base + g * CHUNK, 8)
        pltpu.make_async_copy(
            rows_v.at[j], out_hbm.at[pl.ds(off, CHUNK)], wsems.at[j]
        ).start()

    def wait_write(j):
        pltpu.make_async_copy(
            rows_v.at[j], out_hbm.at[pl.ds(0, CHUNK)], wsems.at[j]
        ).wait()

    for j in range(NBUF - 1):
        start_gather(j, j)

    def body(i, carry):
        g0 = i * NBUF
        for j in range(NBUF):
            g = g0 + j
            wait_gather(j)
            start_write(g, j)
            nxt = g + NBUF - 1
            nj = (j + NBUF - 1) % NBUF

            @pl.when(nxt < NCHUNKS)
            def _():
                if j == 0:
                    @pl.when(i > 0)
                    def _():
                        wait_write(nj)
                else:
                    wait_write(nj)
                start_gather(nxt, nj)

        return carry

    lax.fori_loop(0, NCHUNKS // NBUF, body, 0)

    for j in range(NBUF):
        wait_write(j)


_mesh = plsc.VectorSubcoreMesh(core_axis_name="c", subcore_axis_name="s")

_gather = functools.partial(
    pl.kernel,
    mesh=_mesh,
    out_type=jax.ShapeDtypeStruct((HALF, EMBED_DIM), jnp.float32),
    compiler_params=pltpu.CompilerParams(use_tc_tiling_on_sc=False),
    scratch_types=[
        pltpu.VMEM((NCHUNKS, CHUNK), jnp.int32),
        pltpu.VMEM((NBUF, CHUNK, EMBED_DIM), jnp.float32),
        pltpu.SemaphoreType.DMA((NBUF,)),
        pltpu.SemaphoreType.DMA((NBUF,)),
    ],
)(_gather_body)


@jax.jit
def kernel(x, table):
    xf = x.reshape(-1).astype(jnp.int32).reshape(2, NW, NCHUNKS, CHUNK)
    out0 = _gather(xf[0], table)
    out1 = _gather(xf[1], table)
    half = (BATCH // 2, SEQ, EMBED_DIM)
    return jnp.concatenate(
        [out0.reshape(half), out1.reshape(half)], axis=0
    )


# speedup vs baseline: 1.0761x; 1.0761x over previous
import functools

import jax
import jax.numpy as jnp
from jax import lax
from jax.experimental import pallas as pl
from jax.experimental.pallas import tpu as pltpu
from jax.experimental.pallas import tpu_sc as plsc

VOCAB = 100000
EMBED_DIM = 64
BATCH = 4096
SEQ = 50
N = BATCH * SEQ

_info = plsc.get_sparse_core_info()
NC = _info.num_cores
NS = _info.num_subcores
NW = NC * NS
B_PER_W = N // NW
CHUNK = 128
NCHUNKS = B_PER_W // CHUNK
NBUF = 5


def _gather_body(x_hbm, table_hbm, out_hbm, idx_v, rows_v, gsems, wsems):
    wid = lax.axis_index("s") * NC + lax.axis_index("c")
    base = wid * B_PER_W
    pltpu.sync_copy(x_hbm.at[wid], idx_v)

    def start_gather(g, j):
        pltpu.make_async_copy(
            table_hbm.at[idx_v.at[g]], rows_v.at[j], gsems.at[j]
        ).start()

    def wait_gather(j):
        pltpu.make_async_copy(
            table_hbm.at[idx_v.at[0]], rows_v.at[j], gsems.at[j]
        ).wait()

    def start_write(g, j):
        off = pl.multiple_of(base + g * CHUNK, 8)
        pltpu.make_async_copy(
            rows_v.at[j], out_hbm.at[pl.ds(off, CHUNK)], wsems.at[j]
        ).start()

    def wait_write(j):
        pltpu.make_async_copy(
            rows_v.at[j], out_hbm.at[pl.ds(0, CHUNK)], wsems.at[j]
        ).wait()

    for j in range(NBUF - 1):
        start_gather(j, j)

    def body(i, carry):
        g0 = i * NBUF
        for j in range(NBUF):
            g = g0 + j
            wait_gather(j)
            start_write(g, j)
            nxt = g + NBUF - 1
            nj = (j + NBUF - 1) % NBUF

            @pl.when(nxt < NCHUNKS)
            def _():
                if j == 0:
                    @pl.when(i > 0)
                    def _():
                        wait_write(nj)
                else:
                    wait_write(nj)
                start_gather(nxt, nj)

        return carry

    lax.fori_loop(0, NCHUNKS // NBUF, body, 0)

    for j in range(NBUF):
        wait_write(j)


_mesh = plsc.VectorSubcoreMesh(core_axis_name="c", subcore_axis_name="s")

_gather = functools.partial(
    pl.kernel,
    mesh=_mesh,
    out_type=jax.ShapeDtypeStruct((N, EMBED_DIM), jnp.float32),
    compiler_params=pltpu.CompilerParams(use_tc_tiling_on_sc=False),
    scratch_types=[
        pltpu.VMEM((NCHUNKS, CHUNK), jnp.int32),
        pltpu.VMEM((NBUF, CHUNK, EMBED_DIM), jnp.float32),
        pltpu.SemaphoreType.DMA((NBUF,)),
        pltpu.SemaphoreType.DMA((NBUF,)),
    ],
)(_gather_body)


@jax.jit
def kernel(x, table):
    xf = x.reshape(-1).astype(jnp.int32).reshape(NW, NCHUNKS, CHUNK)
    out = _gather(xf, table)
    return out.reshape(BATCH, SEQ, EMBED_DIM)
